# Initial kernel scaffold; baseline (speedup 1.0000x reference)
import jax
import jax.numpy as jnp
from jax.experimental import pallas as pl


def kernel(x_text, batch_text, x_audio, batch_audio, x_video, batch_video, ei_text_text, ei_text_audio, ei_text_video, ei_audio_text, ei_audio_audio, ei_audio_video, ei_video_text, ei_video_audio, ei_video_video, Win_text, bin_text, Win_audio, bin_audio, Win_video, bin_video, W0_text, W0_audio, W0_video, att0_text_text, att0_text_audio, att0_text_video, att0_audio_text, att0_audio_audio, att0_audio_video, att0_video_text, att0_video_audio, att0_video_video, W1_text, W1_audio, W1_video, att1_text_text, att1_text_audio, att1_text_video, att1_audio_text, att1_audio_audio, att1_audio_video, att1_video_text, att1_video_audio, att1_video_video, Wf1, bf1, Wf2, bf2, Wf3, bf3):
    raise NotImplementedError("write your pallas kernel here")


# speedup vs baseline: 38.8642x; 38.8642x over previous
import functools

import jax
import jax.numpy as jnp
from jax import lax
from jax.experimental import pallas as pl
from jax.experimental.pallas import tpu as pltpu
from jax.experimental.pallas import tpu_sc as plsc

_N = 10000
_E = 65536
_HID = 64
_H = 4
_HD = 16
_B = 64
_RW = 80
_NC = 2
_NS = 16
_CH = 128
_EPT = _E // (_NC * _NS)
_NCHUNK = _EPT // _CH
_NP = 10240
_RPT = _NP // _NS
_RZ = 128


def _embed_body(xt, bt, xa, ba, xv, bv,
                wint, bint, wina, bina, winv, binv,
                w0t, w0a, w0v, ot, oa, ov):
    def one(x_ref, b_ref, win_ref, bin_ref, w0_ref, o_ref):
        x = x_ref[...]
        b = b_ref[...]
        biota = lax.broadcasted_iota(jnp.int32, (1, _B), 1)
        onehot = (b == biota).astype(jnp.float32)
        starts = jnp.sum((b < biota).astype(jnp.float32), axis=0, keepdims=True)
        st = lax.dot_general(onehot, starts, (((1,), (1,)), ((), ())),
                             preferred_element_type=jnp.float32)
        rowi = lax.broadcasted_iota(jnp.int32, (_N, 1), 0).astype(jnp.float32)
        offs = rowi - st
        j = lax.broadcasted_iota(jnp.int32, (1, _HID), 1)
        par = j % 2
        dfull = jnp.exp((j - par).astype(jnp.float32) *
                        (-jnp.log(10000.0) / _HID))
        ang = offs * dfull
        pe = jnp.where(par == 0, jnp.sin(ang), jnp.cos(ang))
        h = lax.dot_general(x, win_ref[...], (((1,), (0,)), ((), ())),
                            preferred_element_type=jnp.float32)
        h = h + bin_ref[...] + pe
        o_ref[pl.ds(0, _N)] = lax.dot_general(
            h, w0_ref[...], (((1,), (0,)), ((), ())),
            preferred_element_type=jnp.float32)

    one(xt, bt, wint, bint, w0t, ot)
    one(xa, ba, wina, bina, w0a, oa)
    one(xv, bv, winv, binv, w0v, ov)


_embed = pl.pallas_call(
    _embed_body,
    out_shape=[jax.ShapeDtypeStruct((_NP, _RW), jnp.float32)] * 3,
)


def _conv_body(htab, ei, att, out,
               eiv, idxs, idxd, hs_v, hd_v, msg_v, att_v, zbuf, acc,
               sem1, sem2):
    core = lax.axis_index("c")
    sub = lax.axis_index("s")

    zv = jnp.zeros((16,), jnp.float32)

    def zb(i, carry):
        for k in range(_RW // _HD):
            zbuf[i, pl.ds(k * _HD, _HD)] = zv
        return carry

    lax.fori_loop(0, _RZ, zb, 0)

    def conn(j, carry):
        jsrc = j // 3
        jdst = j % 3
        soff = jsrc * _NP
        doff = jdst * _NP
        pltpu.sync_copy(att.at[j], att_v)
        for r in range(_RPT // _RZ):
            pltpu.sync_copy(zbuf, acc.at[pl.ds(sub * _RPT + r * _RZ, _RZ)])
        plsc.subcore_barrier()

        def chunk(g, carry2):
            base = (core * _NS + sub) * _EPT + g * _CH
            pltpu.sync_copy(ei.at[j, :, pl.ds(base, _CH)], eiv)

            def off16(k, carry3):
                idxs[pl.ds(k * _HD, _HD)] = eiv[0, pl.ds(k * _HD, _HD)] + soff
                idxd[pl.ds(k * _HD, _HD)] = eiv[1, pl.ds(k * _HD, _HD)] + doff
                return carry3

            lax.fori_loop(0, _CH // _HD, off16, 0)
            cp1 = pltpu.async_copy(htab.at[idxs], hs_v, sem1)
            cp2 = pltpu.async_copy(htab.at[idxd], hd_v, sem2)
            cp1.wait()
            cp2.wait()

            def ebody(e, ecarry):
                den = jnp.zeros((16,), jnp.float32)
                lane = lax.broadcasted_iota(jnp.int32, (16,), 0)
                for h in range(_H):
                    a = hs_v[e, pl.ds(h * _HD, _HD)]
                    t = a + hd_v[e, pl.ds(h * _HD, _HD)]
                    t = jnp.maximum(t, t * 0.2)
                    lg = jnp.sum(t * att_v[h])
                    exv = jnp.exp(jnp.full((16,), lg, jnp.float32))
                    msg_v[e, pl.ds(h * _HD, _HD)] = exv * a
                    den = den + exv * (lane == h).astype(jnp.float32)
                msg_v[e, pl.ds(_HID, _HD)] = den
                return ecarry

            lax.fori_loop(0, _CH, ebody, 0)
            pltpu.sync_copy(msg_v, acc.at[eiv.at[1]], add=True)
            return carry2

        lax.fori_loop(0, _NCHUNK, chunk, 0)
        plsc.subcore_barrier()
        pltpu.sync_copy(acc.at[pl.ds(sub * _RPT, _RPT)],
                        out.at[j, core, pl.ds(sub * _RPT, _RPT)])
        return carry

    lax.fori_loop(0, 9, conn, 0)


@functools.cache
def _get_conv():
  return pl.kernel(
    _conv_body,
    out_type=jax.ShapeDtypeStruct((9, _NC, _NP, _RW), jnp.float32),
    mesh=plsc.VectorSubcoreMesh(core_axis_name="c", subcore_axis_name="s",
                                num_cores=_NC, num_subcores=_NS),
    compiler_params=pltpu.CompilerParams(use_tc_tiling_on_sc=False,
                                         needs_layout_passes=False),
    scratch_types=[
        pltpu.VMEM((2, _CH), jnp.int32),
        pltpu.VMEM((_CH,), jnp.int32),
        pltpu.VMEM((_CH,), jnp.int32),
        pltpu.VMEM((_CH, _RW), jnp.float32),
        pltpu.VMEM((_CH, _RW), jnp.float32),
        pltpu.VMEM((_CH, _RW), jnp.float32),
        pltpu.VMEM((_H, _HD), jnp.float32),
        pltpu.VMEM((_RZ, _RW), jnp.float32),
        pltpu.VMEM_SHARED((_NP, _RW), jnp.float32),
        pltpu.SemaphoreType.DMA,
        pltpu.SemaphoreType.DMA,
    ],
  )


def _split_numden(pj):
    num = pj[0, :, 0:_HID] + pj[1, :, 0:_HID]
    den4 = pj[0, :, _HID:_HID + _H] + pj[1, :, _HID:_HID + _H]
    rows = lax.broadcasted_iota(jnp.int32, (_H, _HID), 0)
    cols = lax.broadcasted_iota(jnp.int32, (_H, _HID), 1) // _HD
    rmat = (rows == cols).astype(jnp.float32)
    dene = lax.dot_general(den4, rmat, (((1,), (0,)), ((), ())),
                           preferred_element_type=jnp.float32)
    return num / (dene + 1e-16)


def _combine_body(p_ref, w_ref, o_ref, xacc):
    t = pl.program_id(0)
    s = t % 3
    out_sd = _split_numden(p_ref[0])

    @pl.when(s == 0)
    def _():
        xacc[...] = out_sd

    @pl.when(s > 0)
    def _():
        xacc[...] = xacc[...] + out_sd

    @pl.when(s == 2)
    def _():
        xv = jnp.maximum(xacc[...] * (1.0 / 3.0), 0.0)
        o_ref[0] = lax.dot_general(xv, w_ref[0], (((1,), (0,)), ((), ())),
                                   preferred_element_type=jnp.float32)


_combine = pl.pallas_call(
    _combine_body,
    grid=(9,),
    in_specs=[
        pl.BlockSpec((1, _NC, _NP, _RW),
                     lambda t: ((t % 3) * 3 + t // 3, 0, 0, 0)),
        pl.BlockSpec((1, _HID, _RW), lambda t: (t // 3, 0, 0)),
    ],
    out_specs=pl.BlockSpec((1, _NP, _RW), lambda t: (t // 3, 0, 0)),
    out_shape=jax.ShapeDtypeStruct((3, _NP, _RW), jnp.float32),
    scratch_shapes=[pltpu.VMEM((_NP, _HID), jnp.float32)],
)


def _readout_body(p_ref, b_ref, wf1, bf1, wf2, bf2, wf3, bf3, o_ref,
                  xacc, ssum, cnt):
    t = pl.program_id(0)
    s = t % 3
    out_sd = _split_numden(p_ref[0])

    @pl.when(t == 0)
    def _():
        ssum[...] = jnp.zeros((_B, _HID), jnp.float32)
        cnt[...] = jnp.zeros((_B, 1), jnp.float32)

    @pl.when(s == 0)
    def _():
        xacc[...] = out_sd

    @pl.when(s > 0)
    def _():
        xacc[...] = xacc[...] + out_sd

    @pl.when(s == 2)
    def _():
        xv = jnp.maximum(xacc[...] * (1.0 / 3.0), 0.0)
        b = b_ref[0]
        biota = lax.broadcasted_iota(jnp.int32, (1, _B), 1)
        onehot = (b == biota).astype(jnp.float32)
        psum = lax.dot_general(onehot, xv, (((0,), (0,)), ((), ())),
                               preferred_element_type=jnp.float32)
        pcnt = lax.dot_general(onehot, jnp.ones((_NP, 1), jnp.float32),
                               (((0,), (0,)), ((), ())),
                               preferred_element_type=jnp.float32)
        ssum[...] = ssum[...] + psum
        cnt[...] = cnt[...] + pcnt

    @pl.when(t == 8)
    def _():
        pooled = ssum[...] / jnp.maximum(cnt[...], 1.0)
        h1 = jnp.maximum(
            lax.dot_general(pooled, wf1[...], (((1,), (0,)), ((), ())),
                            preferred_element_type=jnp.float32) + bf1[...], 0.0)
        h2 = jnp.maximum(
            lax.dot_general(h1, wf2[...], (((1,), (0,)), ((), ())),
                            preferred_element_type=jnp.float32) + bf2[...], 0.0)
        o_ref[...] = lax.dot_general(h2, wf3[...], (((1,), (0,)), ((), ())),
                                     preferred_element_type=jnp.float32) + bf3[...]


_readout = pl.pallas_call(
    _readout_body,
    grid=(9,),
    in_specs=[
        pl.BlockSpec((1, _NC, _NP, _RW),
                     lambda t: ((t % 3) * 3 + t // 3, 0, 0, 0)),
        pl.BlockSpec((1, _NP, 1), lambda t: (t // 3, 0, 0)),
        pl.BlockSpec((_HID, _HID // 4), lambda t: (0, 0)),
        pl.BlockSpec((1, _HID // 4), lambda t: (0, 0)),
        pl.BlockSpec((_HID // 4, _HID // 4), lambda t: (0, 0)),
        pl.BlockSpec((1, _HID // 4), lambda t: (0, 0)),
        pl.BlockSpec((_HID // 4, 1), lambda t: (0, 0)),
        pl.BlockSpec((1, 1), lambda t: (0, 0)),
    ],
    out_specs=pl.BlockSpec((_B, 1), lambda t: (0, 0)),
    out_shape=jax.ShapeDtypeStruct((_B, 1), jnp.float32),
    scratch_shapes=[
        pltpu.VMEM((_NP, _HID), jnp.float32),
        pltpu.VMEM((_B, _HID), jnp.float32),
        pltpu.VMEM((_B, 1), jnp.float32),
    ],
)


def _pad_w(w):
    return jnp.concatenate(
        [w, jnp.zeros((w.shape[0], _RW - w.shape[1]), w.dtype)], axis=1)


def kernel(x_text, batch_text, x_audio, batch_audio, x_video, batch_video,
           ei_text_text, ei_text_audio, ei_text_video,
           ei_audio_text, ei_audio_audio, ei_audio_video,
           ei_video_text, ei_video_audio, ei_video_video,
           Win_text, bin_text, Win_audio, bin_audio, Win_video, bin_video,
           W0_text, W0_audio, W0_video,
           att0_text_text, att0_text_audio, att0_text_video,
           att0_audio_text, att0_audio_audio, att0_audio_video,
           att0_video_text, att0_video_audio, att0_video_video,
           W1_text, W1_audio, W1_video,
           att1_text_text, att1_text_audio, att1_text_video,
           att1_audio_text, att1_audio_audio, att1_audio_video,
           att1_video_text, att1_video_audio, att1_video_video,
           Wf1, bf1, Wf2, bf2, Wf3, bf3):
    f32 = jnp.float32
    bt = batch_text.astype(jnp.int32).reshape(_N, 1)
    ba = batch_audio.astype(jnp.int32).reshape(_N, 1)
    bv = batch_video.astype(jnp.int32).reshape(_N, 1)

    hl0 = _embed(x_text.astype(f32), bt, x_audio.astype(f32), ba,
                 x_video.astype(f32), bv,
                 Win_text, bin_text.reshape(1, _HID),
                 Win_audio, bin_audio.reshape(1, _HID),
                 Win_video, bin_video.reshape(1, _HID),
                 _pad_w(W0_text), _pad_w(W0_audio), _pad_w(W0_video))

    ei = jnp.stack([ei_text_text, ei_text_audio, ei_text_video,
                    ei_audio_text, ei_audio_audio, ei_audio_video,
                    ei_video_text, ei_video_audio, ei_video_video]
                   ).astype(jnp.int32)
    att0 = jnp.stack([att0_text_text, att0_text_audio, att0_text_video,
                      att0_audio_text, att0_audio_audio, att0_audio_video,
                      att0_video_text, att0_video_audio, att0_video_video])
    att1 = jnp.stack([att1_text_text, att1_text_audio, att1_text_video,
                      att1_audio_text, att1_audio_audio, att1_audio_video,
                      att1_video_text, att1_video_audio, att1_video_video])

    conv = _get_conv()
    p0 = conv(jnp.concatenate([hl0[0], hl0[1], hl0[2]]), ei, att0)

    w1 = jnp.stack([_pad_w(W1_text), _pad_w(W1_audio), _pad_w(W1_video)])
    hl1 = _combine(p0, w1)

    p1 = conv(jnp.concatenate([hl1[0], hl1[1], hl1[2]]), ei, att1)

    pad_b = jnp.full((_NP - _N, 1), _B, jnp.int32)
    bstack = jnp.stack([jnp.concatenate([bt, pad_b]),
                        jnp.concatenate([ba, pad_b]),
                        jnp.concatenate([bv, pad_b])])
    o = _readout(p1, bstack,
                 Wf1, bf1.reshape(1, _HID // 4),
                 Wf2, bf2.reshape(1, _HID // 4),
                 Wf3, bf3.reshape(1, 1))
    return o[:, 0]


# speedup vs baseline: 42.5213x; 1.0941x over previous
import functools

import jax
import jax.numpy as jnp
from jax import lax
from jax.experimental import pallas as pl
from jax.experimental.pallas import tpu as pltpu
from jax.experimental.pallas import tpu_sc as plsc

_N = 10000
_E = 65536
_HID = 64
_H = 4
_HD = 16
_B = 64
_RW = 80
_NC = 2
_NS = 16
_CH = 128
_EPT = _E // (_NC * _NS)
_NCHUNK = _EPT // _CH
_NP = 10240
_RPT = _NP // _NS
_RZ = 128


def _embed_body(xt, bt, xa, ba, xv, bv,
                wint, bint, wina, bina, winv, binv,
                w0t, w0a, w0v, ot, oa, ov):
    def one(x_ref, b_ref, win_ref, bin_ref, w0_ref, o_ref):
        x = x_ref[...]
        b = b_ref[...]
        biota = lax.broadcasted_iota(jnp.int32, (1, _B), 1)
        onehot = (b == biota).astype(jnp.float32)
        starts = jnp.sum((b < biota).astype(jnp.float32), axis=0, keepdims=True)
        st = lax.dot_general(onehot, starts, (((1,), (1,)), ((), ())),
                             preferred_element_type=jnp.float32)
        rowi = lax.broadcasted_iota(jnp.int32, (_N, 1), 0).astype(jnp.float32)
        offs = rowi - st
        j = lax.broadcasted_iota(jnp.int32, (1, _HID), 1)
        par = j % 2
        dfull = jnp.exp((j - par).astype(jnp.float32) *
                        (-jnp.log(10000.0) / _HID))
        ang = offs * dfull
        pe = jnp.where(par == 0, jnp.sin(ang), jnp.cos(ang))
        h = lax.dot_general(x, win_ref[...], (((1,), (0,)), ((), ())),
                            preferred_element_type=jnp.float32)
        h = h + bin_ref[...] + pe
        o_ref[pl.ds(0, _N)] = lax.dot_general(
            h, w0_ref[...], (((1,), (0,)), ((), ())),
            preferred_element_type=jnp.float32)

    one(xt, bt, wint, bint, w0t, ot)
    one(xa, ba, wina, bina, w0a, oa)
    one(xv, bv, winv, binv, w0v, ov)


_embed = pl.pallas_call(
    _embed_body,
    out_shape=[jax.ShapeDtypeStruct((_NP, _RW), jnp.float32)] * 3,
)


def _conv_body(htab, ei, att, out,
               eiv0, eiv1, is0, is1, id0, id1, hs0, hs1, hd0, hd1,
               msg_v, att_v, zbuf, acc, g0, g1, g2, g3):
    core = lax.axis_index("c")
    sub = lax.axis_index("s")
    eiv = [eiv0, eiv1]
    idxs = [is0, is1]
    idxd = [id0, id1]
    hs_v = [hs0, hs1]
    hd_v = [hd0, hd1]
    gsem = [g0, g1, g2, g3]

    zv = jnp.zeros((16,), jnp.float32)

    def zb(i, carry):
        for k in range(_RW // _HD):
            zbuf[i, pl.ds(k * _HD, _HD)] = zv
        return carry

    lax.fori_loop(0, _RZ, zb, 0)

    def conn(j, carry):
        jsrc = j // 3
        jdst = j % 3
        soff = jsrc * _NP
        doff = jdst * _NP
        base0 = (core * _NS + sub) * _EPT
        pltpu.sync_copy(att.at[j], att_v)
        for r in range(_RPT // _RZ):
            pltpu.sync_copy(zbuf, acc.at[pl.ds(sub * _RPT + r * _RZ, _RZ)])
        plsc.subcore_barrier()

        def fetch(g, b):
            pltpu.sync_copy(ei.at[j, :, pl.ds(base0 + g * _CH, _CH)], eiv[b])

            def off16(k, c3):
                idxs[b][pl.ds(k * _HD, _HD)] = (
                    eiv[b][0, pl.ds(k * _HD, _HD)] + soff)
                idxd[b][pl.ds(k * _HD, _HD)] = (
                    eiv[b][1, pl.ds(k * _HD, _HD)] + doff)
                return c3

            lax.fori_loop(0, _CH // _HD, off16, 0)
            pltpu.async_copy(htab.at[idxs[b]], hs_v[b], gsem[2 * b])
            pltpu.async_copy(htab.at[idxd[b]], hd_v[b], gsem[2 * b + 1])

        fetch(0, 0)

        def pair(i, carry2):
            gg = i * 2
            for b in (0, 1):
                g = gg + b
                pltpu.make_async_copy(htab.at[idxs[b]], hs_v[b],
                                      gsem[2 * b]).wait()
                pltpu.make_async_copy(htab.at[idxd[b]], hd_v[b],
                                      gsem[2 * b + 1]).wait()

                @pl.when(g + 1 < _NCHUNK)
                def _():
                    fetch(g + 1, 1 - b)

                hsb = hs_v[b]
                hdb = hd_v[b]

                def ebody(e, ecarry):
                    den = jnp.zeros((16,), jnp.float32)
                    lane = lax.broadcasted_iota(jnp.int32, (16,), 0)
                    for h in range(_H):
                        a = hsb[e, pl.ds(h * _HD, _HD)]
                        t = a + hdb[e, pl.ds(h * _HD, _HD)]
                        t = jnp.maximum(t, t * 0.2)
                        lg = jnp.sum(t * att_v[h])
                        exv = jnp.exp(jnp.full((16,), lg, jnp.float32))
                        msg_v[e, pl.ds(h * _HD, _HD)] = exv * a
                        den = den + exv * (lane == h).astype(jnp.float32)
                    msg_v[e, pl.ds(_HID, _HD)] = den
                    return ecarry

                lax.fori_loop(0, _CH, ebody, 0)
                pltpu.sync_copy(msg_v, acc.at[eiv[b].at[1]], add=True)
            return carry2

        lax.fori_loop(0, _NCHUNK // 2, pair, 0)
        plsc.subcore_barrier()
        pltpu.sync_copy(acc.at[pl.ds(sub * _RPT, _RPT)],
                        out.at[j, core, pl.ds(sub * _RPT, _RPT)])
        return carry

    lax.fori_loop(0, 9, conn, 0)


@functools.cache
def _get_conv():
  return pl.kernel(
    _conv_body,
    out_type=jax.ShapeDtypeStruct((9, _NC, _NP, _RW), jnp.float32),
    mesh=plsc.VectorSubcoreMesh(core_axis_name="c", subcore_axis_name="s",
                                num_cores=_NC, num_subcores=_NS),
    compiler_params=pltpu.CompilerParams(use_tc_tiling_on_sc=False,
                                         needs_layout_passes=False),
    scratch_types=[
        pltpu.VMEM((2, _CH), jnp.int32),
        pltpu.VMEM((2, _CH), jnp.int32),
        pltpu.VMEM((_CH,), jnp.int32),
        pltpu.VMEM((_CH,), jnp.int32),
        pltpu.VMEM((_CH,), jnp.int32),
        pltpu.VMEM((_CH,), jnp.int32),
        pltpu.VMEM((_CH, _RW), jnp.float32),
        pltpu.VMEM((_CH, _RW), jnp.float32),
        pltpu.VMEM((_CH, _RW), jnp.float32),
        pltpu.VMEM((_CH, _RW), jnp.float32),
        pltpu.VMEM((_CH, _RW), jnp.float32),
        pltpu.VMEM((_H, _HD), jnp.float32),
        pltpu.VMEM((_RZ, _RW), jnp.float32),
        pltpu.VMEM_SHARED((_NP, _RW), jnp.float32),
        pltpu.SemaphoreType.DMA,
        pltpu.SemaphoreType.DMA,
        pltpu.SemaphoreType.DMA,
        pltpu.SemaphoreType.DMA,
    ],
  )


def _split_numden(pj):
    num = pj[0, :, 0:_HID] + pj[1, :, 0:_HID]
    den4 = pj[0, :, _HID:_HID + _H] + pj[1, :, _HID:_HID + _H]
    rows = lax.broadcasted_iota(jnp.int32, (_H, _HID), 0)
    cols = lax.broadcasted_iota(jnp.int32, (_H, _HID), 1) // _HD
    rmat = (rows == cols).astype(jnp.float32)
    dene = lax.dot_general(den4, rmat, (((1,), (0,)), ((), ())),
                           preferred_element_type=jnp.float32)
    return num / (dene + 1e-16)


def _combine_body(p_ref, w_ref, o_ref, xacc):
    t = pl.program_id(0)
    s = t % 3
    out_sd = _split_numden(p_ref[0])

    @pl.when(s == 0)
    def _():
        xacc[...] = out_sd

    @pl.when(s > 0)
    def _():
        xacc[...] = xacc[...] + out_sd

    @pl.when(s == 2)
    def _():
        xv = jnp.maximum(xacc[...] * (1.0 / 3.0), 0.0)
        o_ref[0] = lax.dot_general(xv, w_ref[0], (((1,), (0,)), ((), ())),
                                   preferred_element_type=jnp.float32)


_combine = pl.pallas_call(
    _combine_body,
    grid=(9,),
    in_specs=[
        pl.BlockSpec((1, _NC, _NP, _RW),
                     lambda t: ((t % 3) * 3 + t // 3, 0, 0, 0)),
        pl.BlockSpec((1, _HID, _RW), lambda t: (t // 3, 0, 0)),
    ],
    out_specs=pl.BlockSpec((1, _NP, _RW), lambda t: (t // 3, 0, 0)),
    out_shape=jax.ShapeDtypeStruct((3, _NP, _RW), jnp.float32),
    scratch_shapes=[pltpu.VMEM((_NP, _HID), jnp.float32)],
)


def _readout_body(p_ref, b_ref, wf1, bf1, wf2, bf2, wf3, bf3, o_ref,
                  xacc, ssum, cnt):
    t = pl.program_id(0)
    s = t % 3
    out_sd = _split_numden(p_ref[0])

    @pl.when(t == 0)
    def _():
        ssum[...] = jnp.zeros((_B, _HID), jnp.float32)
        cnt[...] = jnp.zeros((_B, 1), jnp.float32)

    @pl.when(s == 0)
    def _():
        xacc[...] = out_sd

    @pl.when(s > 0)
    def _():
        xacc[...] = xacc[...] + out_sd

    @pl.when(s == 2)
    def _():
        xv = jnp.maximum(xacc[...] * (1.0 / 3.0), 0.0)
        b = b_ref[0]
        biota = lax.broadcasted_iota(jnp.int32, (1, _B), 1)
        onehot = (b == biota).astype(jnp.float32)
        psum = lax.dot_general(onehot, xv, (((0,), (0,)), ((), ())),
                               preferred_element_type=jnp.float32)
        pcnt = lax.dot_general(onehot, jnp.ones((_NP, 1), jnp.float32),
                               (((0,), (0,)), ((), ())),
                               preferred_element_type=jnp.float32)
        ssum[...] = ssum[...] + psum
        cnt[...] = cnt[...] + pcnt

    @pl.when(t == 8)
    def _():
        pooled = ssum[...] / jnp.maximum(cnt[...], 1.0)
        h1 = jnp.maximum(
            lax.dot_general(pooled, wf1[...], (((1,), (0,)), ((), ())),
                            preferred_element_type=jnp.float32) + bf1[...], 0.0)
        h2 = jnp.maximum(
            lax.dot_general(h1, wf2[...], (((1,), (0,)), ((), ())),
                            preferred_element_type=jnp.float32) + bf2[...], 0.0)
        o_ref[...] = lax.dot_general(h2, wf3[...], (((1,), (0,)), ((), ())),
                                     preferred_element_type=jnp.float32) + bf3[...]


_readout = pl.pallas_call(
    _readout_body,
    grid=(9,),
    in_specs=[
        pl.BlockSpec((1, _NC, _NP, _RW),
                     lambda t: ((t % 3) * 3 + t // 3, 0, 0, 0)),
        pl.BlockSpec((1, _NP, 1), lambda t: (t // 3, 0, 0)),
        pl.BlockSpec((_HID, _HID // 4), lambda t: (0, 0)),
        pl.BlockSpec((1, _HID // 4), lambda t: (0, 0)),
        pl.BlockSpec((_HID // 4, _HID // 4), lambda t: (0, 0)),
        pl.BlockSpec((1, _HID // 4), lambda t: (0, 0)),
        pl.BlockSpec((_HID // 4, 1), lambda t: (0, 0)),
        pl.BlockSpec((1, 1), lambda t: (0, 0)),
    ],
    out_specs=pl.BlockSpec((_B, 1), lambda t: (0, 0)),
    out_shape=jax.ShapeDtypeStruct((_B, 1), jnp.float32),
    scratch_shapes=[
        pltpu.VMEM((_NP, _HID), jnp.float32),
        pltpu.VMEM((_B, _HID), jnp.float32),
        pltpu.VMEM((_B, 1), jnp.float32),
    ],
)


def _pad_w(w):
    return jnp.concatenate(
        [w, jnp.zeros((w.shape[0], _RW - w.shape[1]), w.dtype)], axis=1)


def kernel(x_text, batch_text, x_audio, batch_audio, x_video, batch_video,
           ei_text_text, ei_text_audio, ei_text_video,
           ei_audio_text, ei_audio_audio, ei_audio_video,
           ei_video_text, ei_video_audio, ei_video_video,
           Win_text, bin_text, Win_audio, bin_audio, Win_video, bin_video,
           W0_text, W0_audio, W0_video,
           att0_text_text, att0_text_audio, att0_text_video,
           att0_audio_text, att0_audio_audio, att0_audio_video,
           att0_video_text, att0_video_audio, att0_video_video,
           W1_text, W1_audio, W1_video,
           att1_text_text, att1_text_audio, att1_text_video,
           att1_audio_text, att1_audio_audio, att1_audio_video,
           att1_video_text, att1_video_audio, att1_video_video,
           Wf1, bf1, Wf2, bf2, Wf3, bf3):
    f32 = jnp.float32
    bt = batch_text.astype(jnp.int32).reshape(_N, 1)
    ba = batch_audio.astype(jnp.int32).reshape(_N, 1)
    bv = batch_video.astype(jnp.int32).reshape(_N, 1)

    hl0 = _embed(x_text.astype(f32), bt, x_audio.astype(f32), ba,
                 x_video.astype(f32), bv,
                 Win_text, bin_text.reshape(1, _HID),
                 Win_audio, bin_audio.reshape(1, _HID),
                 Win_video, bin_video.reshape(1, _HID),
                 _pad_w(W0_text), _pad_w(W0_audio), _pad_w(W0_video))

    ei = jnp.stack([ei_text_text, ei_text_audio, ei_text_video,
                    ei_audio_text, ei_audio_audio, ei_audio_video,
                    ei_video_text, ei_video_audio, ei_video_video]
                   ).astype(jnp.int32)
    att0 = jnp.stack([att0_text_text, att0_text_audio, att0_text_video,
                      att0_audio_text, att0_audio_audio, att0_audio_video,
                      att0_video_text, att0_video_audio, att0_video_video])
    att1 = jnp.stack([att1_text_text, att1_text_audio, att1_text_video,
                      att1_audio_text, att1_audio_audio, att1_audio_video,
                      att1_video_text, att1_video_audio, att1_video_video])

    conv = _get_conv()
    p0 = conv(jnp.concatenate([hl0[0], hl0[1], hl0[2]]), ei, att0)

    w1 = jnp.stack([_pad_w(W1_text), _pad_w(W1_audio), _pad_w(W1_video)])
    hl1 = _combine(p0, w1)

    p1 = conv(jnp.concatenate([hl1[0], hl1[1], hl1[2]]), ei, att1)

    pad_b = jnp.full((_NP - _N, 1), _B, jnp.int32)
    bstack = jnp.stack([jnp.concatenate([bt, pad_b]),
                        jnp.concatenate([ba, pad_b]),
                        jnp.concatenate([bv, pad_b])])
    o = _readout(p1, bstack,
                 Wf1, bf1.reshape(1, _HID // 4),
                 Wf2, bf2.reshape(1, _HID // 4),
                 Wf3, bf3.reshape(1, 1))
    return o[:, 0]


# speedup vs baseline: 132.5606x; 3.1175x over previous
import functools

import jax
import jax.numpy as jnp
from jax import lax
from jax.experimental import pallas as pl
from jax.experimental.pallas import tpu as pltpu
from jax.experimental.pallas import tpu_sc as plsc

_N = 10000
_E = 65536
_HID = 64
_H = 4
_HD = 16
_B = 64
_RW = 80
_NC = 2
_NS = 16
_CH = 128
_EPT = _E // (_NC * _NS)
_NCHUNK = _EPT // _CH
_NP = 10240
_RPT = _NP // _NS
_RZ = 128


def _embed_body(xt, bt, xa, ba, xv, bv,
                wint, bint, wina, bina, winv, binv,
                w0t, w0a, w0v, ot, oa, ov):
    def one(x_ref, b_ref, win_ref, bin_ref, w0_ref, o_ref):
        x = x_ref[...]
        b = b_ref[...]
        biota = lax.broadcasted_iota(jnp.int32, (1, _B), 1)
        onehot = (b == biota).astype(jnp.float32)
        starts = jnp.sum((b < biota).astype(jnp.float32), axis=0, keepdims=True)
        st = lax.dot_general(onehot, starts, (((1,), (1,)), ((), ())),
                             preferred_element_type=jnp.float32)
        rowi = lax.broadcasted_iota(jnp.int32, (_N, 1), 0).astype(jnp.float32)
        offs = rowi - st
        j = lax.broadcasted_iota(jnp.int32, (1, _HID), 1)
        par = j % 2
        dfull = jnp.exp((j - par).astype(jnp.float32) *
                        (-jnp.log(10000.0) / _HID))
        ang = offs * dfull
        pe = jnp.where(par == 0, jnp.sin(ang), jnp.cos(ang))
        h = lax.dot_general(x, win_ref[...], (((1,), (0,)), ((), ())),
                            preferred_element_type=jnp.float32)
        h = h + bin_ref[...] + pe
        o_ref[pl.ds(0, _N)] = lax.dot_general(
            h, w0_ref[...], (((1,), (0,)), ((), ())),
            preferred_element_type=jnp.float32)

    one(xt, bt, wint, bint, w0t, ot)
    one(xa, ba, wina, bina, w0a, oa)
    one(xv, bv, winv, binv, w0v, ov)


_embed = pl.pallas_call(
    _embed_body,
    out_shape=[jax.ShapeDtypeStruct((_NP, _RW), jnp.float32)] * 3,
)


def _conv_body(htab, ei, att, out,
               eiv0, eiv1, is0, is1, id0, id1, hs0, hs1, hd0, hd1,
               msg_v, att_v, zbuf, acc, g0, g1, g2, g3):
    core = lax.axis_index("c")
    sub = lax.axis_index("s")
    eiv = [eiv0, eiv1]
    idxs = [is0, is1]
    idxd = [id0, id1]
    hs_v = [hs0, hs1]
    hd_v = [hd0, hd1]
    gsem = [g0, g1, g2, g3]

    zv = jnp.zeros((16,), jnp.float32)

    def zb(i, carry):
        for k in range(_RW // _HD):
            zbuf[i, pl.ds(k * _HD, _HD)] = zv
        return carry

    lax.fori_loop(0, _RZ, zb, 0)

    def conn(j, carry):
        jsrc = j // 3
        jdst = j % 3
        soff = jsrc * _NP
        doff = jdst * _NP
        base0 = (core * _NS + sub) * _EPT
        pltpu.sync_copy(att.at[j], att_v)
        for r in range(_RPT // _RZ):
            pltpu.sync_copy(zbuf, acc.at[pl.ds(sub * _RPT + r * _RZ, _RZ)])
        plsc.subcore_barrier()

        def fetch(g, b):
            pltpu.sync_copy(ei.at[j, :, pl.ds(base0 + g * _CH, _CH)], eiv[b])

            def off16(k, c3):
                idxs[b][pl.ds(k * _HD, _HD)] = (
                    eiv[b][0, pl.ds(k * _HD, _HD)] + soff)
                idxd[b][pl.ds(k * _HD, _HD)] = (
                    eiv[b][1, pl.ds(k * _HD, _HD)] + doff)
                return c3

            lax.fori_loop(0, _CH // _HD, off16, 0)
            pltpu.async_copy(htab.at[idxs[b]], hs_v[b], gsem[2 * b])
            pltpu.async_copy(htab.at[idxd[b]], hd_v[b], gsem[2 * b + 1])

        fetch(0, 0)

        def pair(i, carry2):
            gg = i * 2
            for b in (0, 1):
                g = gg + b
                pltpu.make_async_copy(htab.at[idxs[b]], hs_v[b],
                                      gsem[2 * b]).wait()
                pltpu.make_async_copy(htab.at[idxd[b]], hd_v[b],
                                      gsem[2 * b + 1]).wait()

                @pl.when(g + 1 < _NCHUNK)
                def _():
                    fetch(g + 1, 1 - b)

                hsb = hs_v[b]
                hdb = hd_v[b]

                @plsc.parallel_loop(0, _CH, step=1, unroll=4)
                def _(e):
                    den = jnp.zeros((16,), jnp.float32)
                    lane = lax.broadcasted_iota(jnp.int32, (16,), 0)
                    for h in range(_H):
                        a = hsb[e, pl.ds(h * _HD, _HD)]
                        t = a + hdb[e, pl.ds(h * _HD, _HD)]
                        t = jnp.maximum(t, t * 0.2)
                        lg = jnp.sum(t * att_v[h])
                        exv = jnp.exp(jnp.full((16,), lg, jnp.float32))
                        msg_v[e, pl.ds(h * _HD, _HD)] = exv * a
                        den = den + exv * (lane == h).astype(jnp.float32)
                    msg_v[e, pl.ds(_HID, _HD)] = den
                pltpu.sync_copy(msg_v, acc.at[eiv[b].at[1]], add=True)
            return carry2

        lax.fori_loop(0, _NCHUNK // 2, pair, 0)
        plsc.subcore_barrier()
        pltpu.sync_copy(acc.at[pl.ds(sub * _RPT, _RPT)],
                        out.at[j, core, pl.ds(sub * _RPT, _RPT)])
        return carry

    lax.fori_loop(0, 9, conn, 0)


@functools.cache
def _get_conv():
  return pl.kernel(
    _conv_body,
    out_type=jax.ShapeDtypeStruct((9, _NC, _NP, _RW), jnp.float32),
    mesh=plsc.VectorSubcoreMesh(core_axis_name="c", subcore_axis_name="s",
                                num_cores=_NC, num_subcores=_NS),
    compiler_params=pltpu.CompilerParams(use_tc_tiling_on_sc=False,
                                         needs_layout_passes=False),
    scratch_types=[
        pltpu.VMEM((2, _CH), jnp.int32),
        pltpu.VMEM((2, _CH), jnp.int32),
        pltpu.VMEM((_CH,), jnp.int32),
        pltpu.VMEM((_CH,), jnp.int32),
        pltpu.VMEM((_CH,), jnp.int32),
        pltpu.VMEM((_CH,), jnp.int32),
        pltpu.VMEM((_CH, _RW), jnp.float32),
        pltpu.VMEM((_CH, _RW), jnp.float32),
        pltpu.VMEM((_CH, _RW), jnp.float32),
        pltpu.VMEM((_CH, _RW), jnp.float32),
        pltpu.VMEM((_CH, _RW), jnp.float32),
        pltpu.VMEM((_H, _HD), jnp.float32),
        pltpu.VMEM((_RZ, _RW), jnp.float32),
        pltpu.VMEM_SHARED((_NP, _RW), jnp.float32),
        pltpu.SemaphoreType.DMA,
        pltpu.SemaphoreType.DMA,
        pltpu.SemaphoreType.DMA,
        pltpu.SemaphoreType.DMA,
    ],
  )


def _split_numden(pj):
    num = pj[0, :, 0:_HID] + pj[1, :, 0:_HID]
    den4 = pj[0, :, _HID:_HID + _H] + pj[1, :, _HID:_HID + _H]
    rows = lax.broadcasted_iota(jnp.int32, (_H, _HID), 0)
    cols = lax.broadcasted_iota(jnp.int32, (_H, _HID), 1) // _HD
    rmat = (rows == cols).astype(jnp.float32)
    dene = lax.dot_general(den4, rmat, (((1,), (0,)), ((), ())),
                           preferred_element_type=jnp.float32)
    return num / (dene + 1e-16)


def _combine_body(p_ref, w_ref, o_ref, xacc):
    t = pl.program_id(0)
    s = t % 3
    out_sd = _split_numden(p_ref[0])

    @pl.when(s == 0)
    def _():
        xacc[...] = out_sd

    @pl.when(s > 0)
    def _():
        xacc[...] = xacc[...] + out_sd

    @pl.when(s == 2)
    def _():
        xv = jnp.maximum(xacc[...] * (1.0 / 3.0), 0.0)
        o_ref[0] = lax.dot_general(xv, w_ref[0], (((1,), (0,)), ((), ())),
                                   preferred_element_type=jnp.float32)


_combine = pl.pallas_call(
    _combine_body,
    grid=(9,),
    in_specs=[
        pl.BlockSpec((1, _NC, _NP, _RW),
                     lambda t: ((t % 3) * 3 + t // 3, 0, 0, 0)),
        pl.BlockSpec((1, _HID, _RW), lambda t: (t // 3, 0, 0)),
    ],
    out_specs=pl.BlockSpec((1, _NP, _RW), lambda t: (t // 3, 0, 0)),
    out_shape=jax.ShapeDtypeStruct((3, _NP, _RW), jnp.float32),
    scratch_shapes=[pltpu.VMEM((_NP, _HID), jnp.float32)],
)


def _readout_body(p_ref, b_ref, wf1, bf1, wf2, bf2, wf3, bf3, o_ref,
                  xacc, ssum, cnt):
    t = pl.program_id(0)
    s = t % 3
    out_sd = _split_numden(p_ref[0])

    @pl.when(t == 0)
    def _():
        ssum[...] = jnp.zeros((_B, _HID), jnp.float32)
        cnt[...] = jnp.zeros((_B, 1), jnp.float32)

    @pl.when(s == 0)
    def _():
        xacc[...] = out_sd

    @pl.when(s > 0)
    def _():
        xacc[...] = xacc[...] + out_sd

    @pl.when(s == 2)
    def _():
        xv = jnp.maximum(xacc[...] * (1.0 / 3.0), 0.0)
        b = b_ref[0]
        biota = lax.broadcasted_iota(jnp.int32, (1, _B), 1)
        onehot = (b == biota).astype(jnp.float32)
        psum = lax.dot_general(onehot, xv, (((0,), (0,)), ((), ())),
                               preferred_element_type=jnp.float32)
        pcnt = lax.dot_general(onehot, jnp.ones((_NP, 1), jnp.float32),
                               (((0,), (0,)), ((), ())),
                               preferred_element_type=jnp.float32)
        ssum[...] = ssum[...] + psum
        cnt[...] = cnt[...] + pcnt

    @pl.when(t == 8)
    def _():
        pooled = ssum[...] / jnp.maximum(cnt[...], 1.0)
        h1 = jnp.maximum(
            lax.dot_general(pooled, wf1[...], (((1,), (0,)), ((), ())),
                            preferred_element_type=jnp.float32) + bf1[...], 0.0)
        h2 = jnp.maximum(
            lax.dot_general(h1, wf2[...], (((1,), (0,)), ((), ())),
                            preferred_element_type=jnp.float32) + bf2[...], 0.0)
        o_ref[...] = lax.dot_general(h2, wf3[...], (((1,), (0,)), ((), ())),
                                     preferred_element_type=jnp.float32) + bf3[...]


_readout = pl.pallas_call(
    _readout_body,
    grid=(9,),
    in_specs=[
        pl.BlockSpec((1, _NC, _NP, _RW),
                     lambda t: ((t % 3) * 3 + t // 3, 0, 0, 0)),
        pl.BlockSpec((1, _NP, 1), lambda t: (t // 3, 0, 0)),
        pl.BlockSpec((_HID, _HID // 4), lambda t: (0, 0)),
        pl.BlockSpec((1, _HID // 4), lambda t: (0, 0)),
        pl.BlockSpec((_HID // 4, _HID // 4), lambda t: (0, 0)),
        pl.BlockSpec((1, _HID // 4), lambda t: (0, 0)),
        pl.BlockSpec((_HID // 4, 1), lambda t: (0, 0)),
        pl.BlockSpec((1, 1), lambda t: (0, 0)),
    ],
    out_specs=pl.BlockSpec((_B, 1), lambda t: (0, 0)),
    out_shape=jax.ShapeDtypeStruct((_B, 1), jnp.float32),
    scratch_shapes=[
        pltpu.VMEM((_NP, _HID), jnp.float32),
        pltpu.VMEM((_B, _HID), jnp.float32),
        pltpu.VMEM((_B, 1), jnp.float32),
    ],
)


def _pad_w(w):
    return jnp.concatenate(
        [w, jnp.zeros((w.shape[0], _RW - w.shape[1]), w.dtype)], axis=1)


def kernel(x_text, batch_text, x_audio, batch_audio, x_video, batch_video,
           ei_text_text, ei_text_audio, ei_text_video,
           ei_audio_text, ei_audio_audio, ei_audio_video,
           ei_video_text, ei_video_audio, ei_video_video,
           Win_text, bin_text, Win_audio, bin_audio, Win_video, bin_video,
           W0_text, W0_audio, W0_video,
           att0_text_text, att0_text_audio, att0_text_video,
           att0_audio_text, att0_audio_audio, att0_audio_video,
           att0_video_text, att0_video_audio, att0_video_video,
           W1_text, W1_audio, W1_video,
           att1_text_text, att1_text_audio, att1_text_video,
           att1_audio_text, att1_audio_audio, att1_audio_video,
           att1_video_text, att1_video_audio, att1_video_video,
           Wf1, bf1, Wf2, bf2, Wf3, bf3):
    f32 = jnp.float32
    bt = batch_text.astype(jnp.int32).reshape(_N, 1)
    ba = batch_audio.astype(jnp.int32).reshape(_N, 1)
    bv = batch_video.astype(jnp.int32).reshape(_N, 1)

    hl0 = _embed(x_text.astype(f32), bt, x_audio.astype(f32), ba,
                 x_video.astype(f32), bv,
                 Win_text, bin_text.reshape(1, _HID),
                 Win_audio, bin_audio.reshape(1, _HID),
                 Win_video, bin_video.reshape(1, _HID),
                 _pad_w(W0_text), _pad_w(W0_audio), _pad_w(W0_video))

    ei = jnp.stack([ei_text_text, ei_text_audio, ei_text_video,
                    ei_audio_text, ei_audio_audio, ei_audio_video,
                    ei_video_text, ei_video_audio, ei_video_video]
                   ).astype(jnp.int32)
    att0 = jnp.stack([att0_text_text, att0_text_audio, att0_text_video,
                      att0_audio_text, att0_audio_audio, att0_audio_video,
                      att0_video_text, att0_video_audio, att0_video_video])
    att1 = jnp.stack([att1_text_text, att1_text_audio, att1_text_video,
                      att1_audio_text, att1_audio_audio, att1_audio_video,
                      att1_video_text, att1_video_audio, att1_video_video])

    conv = _get_conv()
    p0 = conv(jnp.concatenate([hl0[0], hl0[1], hl0[2]]), ei, att0)

    w1 = jnp.stack([_pad_w(W1_text), _pad_w(W1_audio), _pad_w(W1_video)])
    hl1 = _combine(p0, w1)

    p1 = conv(jnp.concatenate([hl1[0], hl1[1], hl1[2]]), ei, att1)

    pad_b = jnp.full((_NP - _N, 1), _B, jnp.int32)
    bstack = jnp.stack([jnp.concatenate([bt, pad_b]),
                        jnp.concatenate([ba, pad_b]),
                        jnp.concatenate([bv, pad_b])])
    o = _readout(p1, bstack,
                 Wf1, bf1.reshape(1, _HID // 4),
                 Wf2, bf2.reshape(1, _HID // 4),
                 Wf3, bf3.reshape(1, 1))
    return o[:, 0]


# speedup vs baseline: 135.9243x; 1.0254x over previous
import functools

import jax
import jax.numpy as jnp
from jax import lax
from jax.experimental import pallas as pl
from jax.experimental.pallas import tpu as pltpu
from jax.experimental.pallas import tpu_sc as plsc

_N = 10000
_E = 65536
_HID = 64
_H = 4
_HD = 16
_B = 64
_RW = 80
_NC = 2
_NS = 16
_CH = 128
_EPT = _E // (_NC * _NS)
_NCHUNK = _EPT // _CH
_NP = 10240
_RPT = _NP // _NS
_RZ = 128


def _embed_body(xt, bt, xa, ba, xv, bv,
                wint, bint, wina, bina, winv, binv,
                w0t, w0a, w0v, o_ref3):
    def one(x_ref, b_ref, win_ref, bin_ref, w0_ref, obase):
        x = x_ref[...]
        b = b_ref[...]
        biota = lax.broadcasted_iota(jnp.int32, (1, _B), 1)
        onehot = (b == biota).astype(jnp.float32)
        starts = jnp.sum((b < biota).astype(jnp.float32), axis=0, keepdims=True)
        st = lax.dot_general(onehot, starts, (((1,), (1,)), ((), ())),
                             preferred_element_type=jnp.float32)
        rowi = lax.broadcasted_iota(jnp.int32, (_N, 1), 0).astype(jnp.float32)
        offs = rowi - st
        j = lax.broadcasted_iota(jnp.int32, (1, _HID), 1)
        par = j % 2
        dfull = jnp.exp((j - par).astype(jnp.float32) *
                        (-jnp.log(10000.0) / _HID))
        ang = offs * dfull
        pe = jnp.where(par == 0, jnp.sin(ang), jnp.cos(ang))
        h = lax.dot_general(x, win_ref[...], (((1,), (0,)), ((), ())),
                            preferred_element_type=jnp.float32)
        h = h + bin_ref[...] + pe
        o_ref3[pl.ds(obase, _N)] = lax.dot_general(
            h, w0_ref[...], (((1,), (0,)), ((), ())),
            preferred_element_type=jnp.float32)

    one(xt, bt, wint, bint, w0t, 0)
    one(xa, ba, wina, bina, w0a, _NP)
    one(xv, bv, winv, binv, w0v, 2 * _NP)


_embed = pl.pallas_call(
    _embed_body,
    out_shape=jax.ShapeDtypeStruct((3 * _NP, _RW), jnp.float32),
)


def _conv_body(htab, ei, att, out,
               eiv0, eiv1, is0, is1, id0, id1, hs0, hs1, hd0, hd1,
               msg_v, att_v, zbuf, acc, g0, g1, g2, g3):
    core = lax.axis_index("c")
    sub = lax.axis_index("s")
    eiv = [eiv0, eiv1]
    idxs = [is0, is1]
    idxd = [id0, id1]
    hs_v = [hs0, hs1]
    hd_v = [hd0, hd1]
    gsem = [g0, g1, g2, g3]

    zv = jnp.zeros((16,), jnp.float32)

    def zb(i, carry):
        for k in range(_RW // _HD):
            zbuf[i, pl.ds(k * _HD, _HD)] = zv
        return carry

    lax.fori_loop(0, _RZ, zb, 0)

    def conn(j, carry):
        jsrc = j // 3
        jdst = j % 3
        soff = jsrc * _NP
        doff = jdst * _NP
        base0 = (core * _NS + sub) * _EPT
        pltpu.sync_copy(att.at[j], att_v)
        for r in range(_RPT // _RZ):
            pltpu.sync_copy(zbuf, acc.at[pl.ds(sub * _RPT + r * _RZ, _RZ)])
        plsc.subcore_barrier()

        def fetch(g, b):
            pltpu.sync_copy(ei.at[j, :, pl.ds(base0 + g * _CH, _CH)], eiv[b])

            def off16(k, c3):
                idxs[b][pl.ds(k * _HD, _HD)] = (
                    eiv[b][0, pl.ds(k * _HD, _HD)] + soff)
                idxd[b][pl.ds(k * _HD, _HD)] = (
                    eiv[b][1, pl.ds(k * _HD, _HD)] + doff)
                return c3

            lax.fori_loop(0, _CH // _HD, off16, 0)
            pltpu.async_copy(htab.at[idxs[b]], hs_v[b], gsem[2 * b])
            pltpu.async_copy(htab.at[idxd[b]], hd_v[b], gsem[2 * b + 1])

        fetch(0, 0)

        def pair(i, carry2):
            gg = i * 2
            for b in (0, 1):
                g = gg + b
                pltpu.make_async_copy(htab.at[idxs[b]], hs_v[b],
                                      gsem[2 * b]).wait()
                pltpu.make_async_copy(htab.at[idxd[b]], hd_v[b],
                                      gsem[2 * b + 1]).wait()

                @pl.when(g + 1 < _NCHUNK)
                def _():
                    fetch(g + 1, 1 - b)

                hsb = hs_v[b]
                hdb = hd_v[b]

                @plsc.parallel_loop(0, _CH, step=1, unroll=4)
                def _(e):
                    den = jnp.zeros((16,), jnp.float32)
                    lane = lax.broadcasted_iota(jnp.int32, (16,), 0)
                    for h in range(_H):
                        a = hsb[e, pl.ds(h * _HD, _HD)]
                        t = a + hdb[e, pl.ds(h * _HD, _HD)]
                        t = jnp.maximum(t, t * 0.2)
                        lg = jnp.sum(t * att_v[h])
                        exv = jnp.exp(jnp.full((16,), lg, jnp.float32))
                        msg_v[e, pl.ds(h * _HD, _HD)] = exv * a
                        den = den + exv * (lane == h).astype(jnp.float32)
                    msg_v[e, pl.ds(_HID, _HD)] = den
                pltpu.sync_copy(msg_v, acc.at[eiv[b].at[1]], add=True)
            return carry2

        lax.fori_loop(0, _NCHUNK // 2, pair, 0)
        plsc.subcore_barrier()
        pltpu.sync_copy(acc.at[pl.ds(sub * _RPT, _RPT)],
                        out.at[j, core, pl.ds(sub * _RPT, _RPT)])
        return carry

    lax.fori_loop(0, 9, conn, 0)


@functools.cache
def _get_conv():
  return pl.kernel(
    _conv_body,
    out_type=jax.ShapeDtypeStruct((9, _NC, _NP, _RW), jnp.float32),
    mesh=plsc.VectorSubcoreMesh(core_axis_name="c", subcore_axis_name="s",
                                num_cores=_NC, num_subcores=_NS),
    compiler_params=pltpu.CompilerParams(use_tc_tiling_on_sc=False,
                                         needs_layout_passes=False),
    scratch_types=[
        pltpu.VMEM((2, _CH), jnp.int32),
        pltpu.VMEM((2, _CH), jnp.int32),
        pltpu.VMEM((_CH,), jnp.int32),
        pltpu.VMEM((_CH,), jnp.int32),
        pltpu.VMEM((_CH,), jnp.int32),
        pltpu.VMEM((_CH,), jnp.int32),
        pltpu.VMEM((_CH, _RW), jnp.float32),
        pltpu.VMEM((_CH, _RW), jnp.float32),
        pltpu.VMEM((_CH, _RW), jnp.float32),
        pltpu.VMEM((_CH, _RW), jnp.float32),
        pltpu.VMEM((_CH, _RW), jnp.float32),
        pltpu.VMEM((_H, _HD), jnp.float32),
        pltpu.VMEM((_RZ, _RW), jnp.float32),
        pltpu.VMEM_SHARED((_NP, _RW), jnp.float32),
        pltpu.SemaphoreType.DMA,
        pltpu.SemaphoreType.DMA,
        pltpu.SemaphoreType.DMA,
        pltpu.SemaphoreType.DMA,
    ],
  )


def _split_numden(pj):
    num = pj[0, :, 0:_HID] + pj[1, :, 0:_HID]
    den4 = pj[0, :, _HID:_HID + _H] + pj[1, :, _HID:_HID + _H]
    rows = lax.broadcasted_iota(jnp.int32, (_H, _HID), 0)
    cols = lax.broadcasted_iota(jnp.int32, (_H, _HID), 1) // _HD
    rmat = (rows == cols).astype(jnp.float32)
    dene = lax.dot_general(den4, rmat, (((1,), (0,)), ((), ())),
                           preferred_element_type=jnp.float32)
    return num / (dene + 1e-16)


def _combine_body(p_ref, w_ref, o_ref, xacc):
    t = pl.program_id(0)
    s = t % 3
    out_sd = _split_numden(p_ref[0])

    @pl.when(s == 0)
    def _():
        xacc[...] = out_sd

    @pl.when(s > 0)
    def _():
        xacc[...] = xacc[...] + out_sd

    @pl.when(s == 2)
    def _():
        xv = jnp.maximum(xacc[...] * (1.0 / 3.0), 0.0)
        o_ref[0] = lax.dot_general(xv, w_ref[0], (((1,), (0,)), ((), ())),
                                   preferred_element_type=jnp.float32)


_combine = pl.pallas_call(
    _combine_body,
    grid=(9,),
    in_specs=[
        pl.BlockSpec((1, _NC, _NP, _RW),
                     lambda t: ((t % 3) * 3 + t // 3, 0, 0, 0)),
        pl.BlockSpec((1, _HID, _RW), lambda t: (t // 3, 0, 0)),
    ],
    out_specs=pl.BlockSpec((1, _NP, _RW), lambda t: (t // 3, 0, 0)),
    out_shape=jax.ShapeDtypeStruct((3, _NP, _RW), jnp.float32),
    scratch_shapes=[pltpu.VMEM((_NP, _HID), jnp.float32)],
)


def _readout_body(p_ref, b_ref, wf1, bf1, wf2, bf2, wf3, bf3, o_ref,
                  xacc, ssum, cnt):
    t = pl.program_id(0)
    s = t % 3
    out_sd = _split_numden(p_ref[0])

    @pl.when(t == 0)
    def _():
        ssum[...] = jnp.zeros((_B, _HID), jnp.float32)
        cnt[...] = jnp.zeros((_B, 1), jnp.float32)

    @pl.when(s == 0)
    def _():
        xacc[...] = out_sd

    @pl.when(s > 0)
    def _():
        xacc[...] = xacc[...] + out_sd

    @pl.when(s == 2)
    def _():
        xv = jnp.maximum(xacc[...] * (1.0 / 3.0), 0.0)
        b = b_ref[0]
        biota = lax.broadcasted_iota(jnp.int32, (1, _B), 1)
        onehot = (b == biota).astype(jnp.float32)
        psum = lax.dot_general(onehot, xv, (((0,), (0,)), ((), ())),
                               preferred_element_type=jnp.float32)
        pcnt = lax.dot_general(onehot, jnp.ones((_NP, 1), jnp.float32),
                               (((0,), (0,)), ((), ())),
                               preferred_element_type=jnp.float32)
        ssum[...] = ssum[...] + psum
        cnt[...] = cnt[...] + pcnt

    @pl.when(t == 8)
    def _():
        pooled = ssum[...] / jnp.maximum(cnt[...], 1.0)
        h1 = jnp.maximum(
            lax.dot_general(pooled, wf1[...], (((1,), (0,)), ((), ())),
                            preferred_element_type=jnp.float32) + bf1[...], 0.0)
        h2 = jnp.maximum(
            lax.dot_general(h1, wf2[...], (((1,), (0,)), ((), ())),
                            preferred_element_type=jnp.float32) + bf2[...], 0.0)
        o_ref[...] = lax.dot_general(h2, wf3[...], (((1,), (0,)), ((), ())),
                                     preferred_element_type=jnp.float32) + bf3[...]


_readout = pl.pallas_call(
    _readout_body,
    grid=(9,),
    in_specs=[
        pl.BlockSpec((1, _NC, _NP, _RW),
                     lambda t: ((t % 3) * 3 + t // 3, 0, 0, 0)),
        pl.BlockSpec((1, _NP, 1), lambda t: (t // 3, 0, 0)),
        pl.BlockSpec((_HID, _HID // 4), lambda t: (0, 0)),
        pl.BlockSpec((1, _HID // 4), lambda t: (0, 0)),
        pl.BlockSpec((_HID // 4, _HID // 4), lambda t: (0, 0)),
        pl.BlockSpec((1, _HID // 4), lambda t: (0, 0)),
        pl.BlockSpec((_HID // 4, 1), lambda t: (0, 0)),
        pl.BlockSpec((1, 1), lambda t: (0, 0)),
    ],
    out_specs=pl.BlockSpec((_B, 1), lambda t: (0, 0)),
    out_shape=jax.ShapeDtypeStruct((_B, 1), jnp.float32),
    scratch_shapes=[
        pltpu.VMEM((_NP, _HID), jnp.float32),
        pltpu.VMEM((_B, _HID), jnp.float32),
        pltpu.VMEM((_B, 1), jnp.float32),
    ],
)


def _pad_w(w):
    return jnp.concatenate(
        [w, jnp.zeros((w.shape[0], _RW - w.shape[1]), w.dtype)], axis=1)


def kernel(x_text, batch_text, x_audio, batch_audio, x_video, batch_video,
           ei_text_text, ei_text_audio, ei_text_video,
           ei_audio_text, ei_audio_audio, ei_audio_video,
           ei_video_text, ei_video_audio, ei_video_video,
           Win_text, bin_text, Win_audio, bin_audio, Win_video, bin_video,
           W0_text, W0_audio, W0_video,
           att0_text_text, att0_text_audio, att0_text_video,
           att0_audio_text, att0_audio_audio, att0_audio_video,
           att0_video_text, att0_video_audio, att0_video_video,
           W1_text, W1_audio, W1_video,
           att1_text_text, att1_text_audio, att1_text_video,
           att1_audio_text, att1_audio_audio, att1_audio_video,
           att1_video_text, att1_video_audio, att1_video_video,
           Wf1, bf1, Wf2, bf2, Wf3, bf3):
    f32 = jnp.float32
    bt = batch_text.astype(jnp.int32).reshape(_N, 1)
    ba = batch_audio.astype(jnp.int32).reshape(_N, 1)
    bv = batch_video.astype(jnp.int32).reshape(_N, 1)

    hl0 = _embed(x_text.astype(f32), bt, x_audio.astype(f32), ba,
                 x_video.astype(f32), bv,
                 Win_text, bin_text.reshape(1, _HID),
                 Win_audio, bin_audio.reshape(1, _HID),
                 Win_video, bin_video.reshape(1, _HID),
                 _pad_w(W0_text), _pad_w(W0_audio), _pad_w(W0_video))

    ei = jnp.stack([ei_text_text, ei_text_audio, ei_text_video,
                    ei_audio_text, ei_audio_audio, ei_audio_video,
                    ei_video_text, ei_video_audio, ei_video_video]
                   ).astype(jnp.int32)
    att0 = jnp.stack([att0_text_text, att0_text_audio, att0_text_video,
                      att0_audio_text, att0_audio_audio, att0_audio_video,
                      att0_video_text, att0_video_audio, att0_video_video])
    att1 = jnp.stack([att1_text_text, att1_text_audio, att1_text_video,
                      att1_audio_text, att1_audio_audio, att1_audio_video,
                      att1_video_text, att1_video_audio, att1_video_video])

    conv = _get_conv()
    p0 = conv(hl0, ei, att0)

    w1 = jnp.stack([_pad_w(W1_text), _pad_w(W1_audio), _pad_w(W1_video)])
    hl1 = _combine(p0, w1)

    p1 = conv(hl1.reshape(3 * _NP, _RW), ei, att1)

    pad_b = jnp.full((_NP - _N, 1), _B, jnp.int32)
    bstack = jnp.stack([jnp.concatenate([bt, pad_b]),
                        jnp.concatenate([ba, pad_b]),
                        jnp.concatenate([bv, pad_b])])
    o = _readout(p1, bstack,
                 Wf1, bf1.reshape(1, _HID // 4),
                 Wf2, bf2.reshape(1, _HID // 4),
                 Wf3, bf3.reshape(1, 1))
    return o[:, 0]
